# Initial kernel scaffold; baseline (speedup 1.0000x reference)
import jax
import jax.numpy as jnp
from jax.experimental import pallas as pl


def kernel(features, emb, Wih_f, Whh_f, bih_f, bhh_f, Wih_b, Whh_b, bih_b, bhh_b, Wc, bc, Wh, bh):
    raise NotImplementedError("write your pallas kernel here")


# speedup vs baseline: 33.0822x; 33.0822x over previous
import functools

import jax
import jax.numpy as jnp
from jax.experimental import pallas as pl
from jax.experimental.pallas import tpu as pltpu

N_FEAT = 26
B = 1024
L = 50
CARD = 101
EDIM = 8
D = N_FEAT * EDIM
H = 128
G3 = 3 * H
TOP = 32
ROWS = B * L

_INTERPRET = False


def _embed_proj_kernel(feat_ref, emb_ref, wx_ref, bx_ref, gif_ref, gib_ref, x_ref):
    R = feat_ref.shape[0]
    for f in range(N_FEAT):
        idx = feat_ref[:, f : f + 1]
        iota = jax.lax.broadcasted_iota(jnp.int32, (R, CARD), 1)
        oh = (idx == iota).astype(jnp.float32)
        x_ref[:, f * EDIM:(f + 1) * EDIM] = jnp.dot(
            oh, emb_ref[f], preferred_element_type=jnp.float32)
    gi = jnp.dot(x_ref[...], wx_ref[...], preferred_element_type=jnp.float32) + bx_ref[0]
    gif_ref[...] = gi[:, :G3]
    gib_ref[...] = gi[:, G3:]


def _gru_kernel(gif_ref, gib_ref, whhf_ref, bhhf_ref, whhb_ref, bhhb_ref,
                wc_ref, bc_ref, wh_ref, bh_ref, out_ref,
                hf, hb, mxf, mxb, smf, smb):
    l = pl.program_id(0)

    @pl.when(l == 0)
    def _init():
        zeros = jnp.zeros((B, H), dtype=jnp.float32)
        neg = jnp.full((B, H), -1e30, dtype=jnp.float32)
        hf[...] = zeros
        hb[...] = zeros
        smf[...] = zeros
        smb[...] = zeros
        mxf[...] = neg
        mxb[...] = neg

    def step(gi, h, whhT_ref, bhh_ref):
        gh = jnp.dot(h, whhT_ref[...], preferred_element_type=jnp.float32) + bhh_ref[0]
        r = jax.nn.sigmoid(gi[:, :H] + gh[:, :H])
        z = jax.nn.sigmoid(gi[:, H:2 * H] + gh[:, H:2 * H])
        n = jnp.tanh(gi[:, 2 * H:] + r * gh[:, 2 * H:])
        return (1.0 - z) * n + z * h

    hf_new = step(gif_ref[...], hf[...], whhf_ref, bhhf_ref)
    hb_new = step(gib_ref[...], hb[...], whhb_ref, bhhb_ref)
    hf[...] = hf_new
    hb[...] = hb_new
    mxf[...] = jnp.maximum(mxf[...], hf_new)
    mxb[...] = jnp.maximum(mxb[...], hb_new)
    smf[...] = smf[...] + hf_new
    smb[...] = smb[...] + hb_new

    @pl.when(l == L - 1)
    def _head():
        inv_l = 1.0 / L
        combined = jnp.concatenate(
            [hf[...], hb[...], mxf[...], mxb[...], smf[...] * inv_l, smb[...] * inv_l],
            axis=1)
        act = jax.nn.relu(
            jnp.dot(combined, wc_ref[...], preferred_element_type=jnp.float32)
            + bc_ref[0])
        out_ref[...] = jnp.sum(act * wh_ref[0][None, :], axis=1, keepdims=True) + bh_ref[0]


def kernel(features, emb, Wih_f, Whh_f, bih_f, bhh_f, Wih_b, Whh_b, bih_b, bhh_b,
           Wc, bc, Wh, bh):
    feat3 = jnp.transpose(features, (2, 1, 0)).reshape(ROWS, N_FEAT)
    Wx = jnp.concatenate([Wih_f, Wih_b], axis=0).T
    bx = jnp.concatenate([bih_f, bih_b]).reshape(1, 2 * G3)
    WhhfT = Whh_f.T
    WhhbT = Whh_b.T
    bhhf2 = bhh_f.reshape(1, G3)
    bhhb2 = bhh_b.reshape(1, G3)
    WcT = Wc.T
    bc2 = bc.reshape(1, TOP)
    bh2 = bh.reshape(1, 1)

    RB = 1600
    n_blocks = ROWS // RB
    gif, gib = pl.pallas_call(
        _embed_proj_kernel,
        grid=(n_blocks,),
        in_specs=[
            pl.BlockSpec((RB, N_FEAT), lambda i: (i, 0)),
            pl.BlockSpec((N_FEAT, CARD, EDIM), lambda i: (0, 0, 0)),
            pl.BlockSpec((D, 2 * G3), lambda i: (0, 0)),
            pl.BlockSpec((1, 2 * G3), lambda i: (0, 0)),
        ],
        out_specs=[
            pl.BlockSpec((RB, G3), lambda i: (i, 0)),
            pl.BlockSpec((RB, G3), lambda i: (i, 0)),
        ],
        out_shape=[
            jax.ShapeDtypeStruct((ROWS, G3), jnp.float32),
            jax.ShapeDtypeStruct((ROWS, G3), jnp.float32),
        ],
        scratch_shapes=[pltpu.VMEM((RB, D), jnp.float32)],
        interpret=_INTERPRET,
    )(feat3, emb, Wx, bx)

    out = pl.pallas_call(
        _gru_kernel,
        grid=(L,),
        in_specs=[
            pl.BlockSpec((B, G3), lambda l: (l, 0)),
            pl.BlockSpec((B, G3), lambda l: (L - 1 - l, 0)),
            pl.BlockSpec((H, G3), lambda l: (0, 0)),
            pl.BlockSpec((1, G3), lambda l: (0, 0)),
            pl.BlockSpec((H, G3), lambda l: (0, 0)),
            pl.BlockSpec((1, G3), lambda l: (0, 0)),
            pl.BlockSpec((6 * H, TOP), lambda l: (0, 0)),
            pl.BlockSpec((1, TOP), lambda l: (0, 0)),
            pl.BlockSpec((1, TOP), lambda l: (0, 0)),
            pl.BlockSpec((1, 1), lambda l: (0, 0)),
        ],
        out_specs=pl.BlockSpec((B, 1), lambda l: (0, 0)),
        out_shape=jax.ShapeDtypeStruct((B, 1), jnp.float32),
        scratch_shapes=[pltpu.VMEM((B, H), jnp.float32)] * 6,
        compiler_params=pltpu.CompilerParams(
            dimension_semantics=("arbitrary",)),
        interpret=_INTERPRET,
    )(gif, gib, WhhfT, bhhf2, WhhbT, bhhb2, WcT, bc2, Wh, bh2)
    return out
